# baseline (speedup 1.0000x reference)
import jax
import jax.numpy as jnp
from jax import lax
from jax.experimental import pallas as pl
from jax.experimental.pallas import tpu as pltpu

NSYM = 1000
NDRUG = 1000000
DIM = 32
BATCH = 2
SEQ = 64
CP = 3328
NB = 301
NPAD = CP * NB


def _sset_body(symsT_ref, semb_ref, w_ref, b_ref, out_ref):
    for b in range(BATCH):
        ids = symsT_ref[:, b:b + 1]
        iota = lax.broadcasted_iota(jnp.int32, (SEQ, NSYM + 1), 1)
        oh = (iota == ids).astype(jnp.float32)
        emb = jnp.dot(oh, semb_ref[...], precision=lax.Precision.HIGHEST,
                      preferred_element_type=jnp.float32)
        a = jnp.dot(emb.astype(jnp.bfloat16), w_ref[...].astype(jnp.bfloat16),
                    preferred_element_type=jnp.float32) + b_ref[0, 0]
        m = jnp.max(a, axis=0, keepdims=True)
        e = jnp.exp(a - m)
        wts = e / jnp.sum(e, axis=0, keepdims=True)
        pooled = jnp.sum(wts * emb, axis=0, keepdims=True)
        n = jnp.sqrt(jnp.sum(pooled * pooled, axis=1, keepdims=True))
        out_ref[b:b + 1, :] = pooled / jnp.maximum(n, 1e-12)


def _compute_sset(symsT, sym_emb, attn_w, attn_b2):
    return pl.pallas_call(
        _sset_body,
        out_shape=jax.ShapeDtypeStruct((BATCH, DIM), jnp.float32),
    )(symsT, sym_emb, attn_w, attn_b2)


def _score_body(sset_ref, demb_ref, rs_ref, out_ref, acc_ref):
    i = pl.program_id(0)
    x = demb_ref[...]
    n = jnp.sqrt(jnp.sum(x * x, axis=1, keepdims=True))
    xn = x / jnp.maximum(n, 1e-12)
    dn = (((1,), (1,)), ((), ()))
    s2 = lax.dot_general(sset_ref[...].astype(jnp.bfloat16),
                         xn.astype(jnp.bfloat16), dn,
                         preferred_element_type=jnp.float32)
    sc = jnp.clip(s2, -10.0, 10.0)
    prob = jax.nn.sigmoid(sc)
    s = jnp.where(prob > 0.5, prob, 0.0)
    rid = lax.broadcasted_iota(jnp.int32, (BATCH, CP), 1) + i * CP
    s = jnp.where((rid >= 1) & (rid <= NDRUG), s, 0.0)
    out_ref[...] = s
    colsum = jnp.sum(s, axis=0, keepdims=True)
    part = jnp.sum(colsum * rs_ref[0])

    @pl.when(i == 0)
    def _():
        acc_ref[...] = jnp.zeros((1, 1), jnp.float32)

    acc_ref[...] = acc_ref[...] + part


def _score_call(s_set, drug_emb, rs_pad):
    return pl.pallas_call(
        _score_body,
        grid=(NB,),
        in_specs=[
            pl.BlockSpec((BATCH, DIM), lambda i: (0, 0)),
            pl.BlockSpec((CP, DIM), lambda i: (i, 0)),
            pl.BlockSpec((1, 1, CP), lambda i: (i, 0, 0)),
        ],
        out_specs=[
            pl.BlockSpec((BATCH, CP), lambda i: (0, i)),
            pl.BlockSpec((1, 1), lambda i: (0, 0)),
        ],
        out_shape=[
            jax.ShapeDtypeStruct((BATCH, NPAD), jnp.float32),
            jax.ShapeDtypeStruct((1, 1), jnp.float32),
        ],
    )(s_set, drug_emb, rs_pad)


def _row_sums(ddi_indices, ddi_values):
    return jax.ops.segment_sum(ddi_values, ddi_indices[0], num_segments=NDRUG)


def kernel(syms, drugs, similar_idx, sym_emb, drug_emb, attn_w, attn_b,
           ddi_indices, ddi_values):
    del drugs, similar_idx
    s_set = _compute_sset(syms.T, sym_emb, attn_w, attn_b.reshape(1, 1))
    row_sums = _row_sums(ddi_indices, ddi_values)
    rs_pad = jnp.pad(row_sums, (1, NPAD - NDRUG - 1)).reshape(NB, 1, CP)
    out_full, acc = _score_call(s_set, drug_emb, rs_pad)
    scores = lax.slice(out_full, (0, 1), (BATCH, NDRUG + 1))
    batch_neg = 1e-6 * acc[0, 0]
    scores_aug = jnp.float32(0.0)
    return (scores, scores_aug, batch_neg)


# speedup vs baseline: 5.8969x; 5.8969x over previous
import functools

import jax
import jax.numpy as jnp
from jax import lax
from jax.experimental import pallas as pl
from jax.experimental.pallas import tpu as pltpu
from jax.experimental.pallas import tpu_sc as plsc

NSYM = 1000
NDRUG = 1000000
DIM = 32
BATCH = 2
SEQ = 64
CP = 3328
NB = 301
NPAD = CP * NB


def _sset_body(symsT_ref, semb_ref, w_ref, b_ref, out_ref):
    for b in range(BATCH):
        ids = symsT_ref[:, b:b + 1]
        iota = lax.broadcasted_iota(jnp.int32, (SEQ, NSYM + 1), 1)
        oh = (iota == ids).astype(jnp.float32)
        emb = jnp.dot(oh, semb_ref[...], precision=lax.Precision.HIGHEST,
                      preferred_element_type=jnp.float32)
        a = jnp.dot(emb.astype(jnp.bfloat16), w_ref[...].astype(jnp.bfloat16),
                    preferred_element_type=jnp.float32) + b_ref[0, 0]
        m = jnp.max(a, axis=0, keepdims=True)
        e = jnp.exp(a - m)
        wts = e / jnp.sum(e, axis=0, keepdims=True)
        pooled = jnp.sum(wts * emb, axis=0, keepdims=True)
        n = jnp.sqrt(jnp.sum(pooled * pooled, axis=1, keepdims=True))
        out_ref[b:b + 1, :] = pooled / jnp.maximum(n, 1e-12)


def _compute_sset(symsT, sym_emb, attn_w, attn_b2):
    return pl.pallas_call(
        _sset_body,
        out_shape=jax.ShapeDtypeStruct((BATCH, DIM), jnp.float32),
    )(symsT, sym_emb, attn_w, attn_b2)


def _score_body(sset_ref, demb_ref, rs_ref, out_ref, acc_ref):
    i = pl.program_id(0)
    x = demb_ref[...]
    n = jnp.sqrt(jnp.sum(x * x, axis=1, keepdims=True))
    xn = x / jnp.maximum(n, 1e-12)
    dn = (((1,), (1,)), ((), ()))
    s2 = lax.dot_general(sset_ref[...].astype(jnp.bfloat16),
                         xn.astype(jnp.bfloat16), dn,
                         preferred_element_type=jnp.float32)
    sc = jnp.clip(s2, -10.0, 10.0)
    prob = jax.nn.sigmoid(sc)
    s = jnp.where(prob > 0.5, prob, 0.0)
    rid = lax.broadcasted_iota(jnp.int32, (BATCH, CP), 1) + i * CP
    s = jnp.where((rid >= 1) & (rid <= NDRUG), s, 0.0)
    out_ref[...] = s
    colsum = jnp.sum(s, axis=0, keepdims=True)
    part = jnp.sum(colsum * rs_ref[0])

    @pl.when(i == 0)
    def _():
        acc_ref[...] = jnp.zeros((1, 1), jnp.float32)

    acc_ref[...] = acc_ref[...] + part


def _score_call(s_set, drug_emb, rs_pad):
    return pl.pallas_call(
        _score_body,
        grid=(NB,),
        in_specs=[
            pl.BlockSpec((BATCH, DIM), lambda i: (0, 0)),
            pl.BlockSpec((CP, DIM), lambda i: (i, 0)),
            pl.BlockSpec((1, 1, CP), lambda i: (i, 0, 0)),
        ],
        out_specs=[
            pl.BlockSpec((BATCH, CP), lambda i: (0, i)),
            pl.BlockSpec((1, 1), lambda i: (0, 0)),
        ],
        out_shape=[
            jax.ShapeDtypeStruct((BATCH, NPAD), jnp.float32),
            jax.ShapeDtypeStruct((1, 1), jnp.float32),
        ],
    )(s_set, drug_emb, rs_pad)


NW = 32
RW = 496
GW = 16
NGW = RW // GW
TOTP = NW * RW * 128
SLAB = 62528
NBINS = 16 * SLAB


def _hist_body(idx_hbm, val_hbm, out_hbm, idx_v, val_v, zbuf, hist):
    c = lax.axis_index("c")
    s = lax.axis_index("s")
    w = s * 2 + c

    def zb(i, carry):
        zbuf[pl.ds(i * 16, 16)] = jnp.zeros((16,), jnp.float32)
        return carry

    lax.fori_loop(0, SLAB // 16, zb, 0)
    pltpu.sync_copy(zbuf, hist.at[pl.ds(s * SLAB, SLAB)])
    plsc.subcore_barrier()

    def grp(g, carry):
        pltpu.sync_copy(idx_hbm.at[w, pl.ds(g * GW, GW)], idx_v)
        pltpu.sync_copy(val_hbm.at[w, pl.ds(g * GW, GW)], val_v)
        for j in range(GW):
            pltpu.sync_copy(val_v.at[j], hist.at[idx_v.at[j]], add=True)
        return carry

    lax.fori_loop(0, NGW, grp, 0)
    plsc.subcore_barrier()
    pltpu.sync_copy(hist.at[pl.ds(s * SLAB, SLAB)], zbuf)
    pltpu.sync_copy(zbuf, out_hbm.at[c, s])


_hist_call = functools.partial(
    pl.kernel,
    out_type=jax.ShapeDtypeStruct((2, 16, SLAB), jnp.float32),
    mesh=plsc.VectorSubcoreMesh(core_axis_name="c", subcore_axis_name="s"),
    scratch_types=[
        pltpu.VMEM((GW, 128), jnp.int32),
        pltpu.VMEM((GW, 128), jnp.float32),
        pltpu.VMEM((SLAB,), jnp.float32),
        pltpu.VMEM_SHARED((NBINS,), jnp.float32),
    ],
)(_hist_body)


def _row_sums_shifted(ddi_indices, ddi_values):
    idxp = jnp.pad(ddi_indices[0] + 1, (0, TOTP - ddi_indices.shape[1]))
    valp = jnp.pad(ddi_values, (0, TOTP - ddi_values.shape[0]))
    parts = _hist_call(idxp.reshape(NW, RW, 128), valp.reshape(NW, RW, 128))
    return (parts[0] + parts[1]).reshape(NBINS)


def kernel(syms, drugs, similar_idx, sym_emb, drug_emb, attn_w, attn_b,
           ddi_indices, ddi_values):
    del drugs, similar_idx
    s_set = _compute_sset(syms.T, sym_emb, attn_w, attn_b.reshape(1, 1))
    rs = _row_sums_shifted(ddi_indices, ddi_values)
    rs_pad = jnp.pad(rs, (0, NPAD - NBINS)).reshape(NB, 1, CP)
    out_full, acc = _score_call(s_set, drug_emb, rs_pad)
    scores = lax.slice(out_full, (0, 1), (BATCH, NDRUG + 1))
    batch_neg = 1e-6 * acc[0, 0]
    scores_aug = jnp.float32(0.0)
    return (scores, scores_aug, batch_neg)


# speedup vs baseline: 6.9920x; 1.1857x over previous
import functools

import jax
import jax.numpy as jnp
from jax import lax
from jax.experimental import pallas as pl
from jax.experimental.pallas import tpu as pltpu
from jax.experimental.pallas import tpu_sc as plsc

NSYM = 1000
NDRUG = 1000000
DIM = 32
BATCH = 2
SEQ = 64
CP = 16128
NB = 63
NPAD = CP * NB


def _sset_body(symsT_ref, semb_ref, w_ref, b_ref, out_ref):
    for b in range(BATCH):
        ids = symsT_ref[:, b:b + 1]
        iota = lax.broadcasted_iota(jnp.int32, (SEQ, NSYM + 1), 1)
        oh = (iota == ids).astype(jnp.float32)
        emb = jnp.dot(oh, semb_ref[...], precision=lax.Precision.HIGHEST,
                      preferred_element_type=jnp.float32)
        a = jnp.dot(emb.astype(jnp.bfloat16), w_ref[...].astype(jnp.bfloat16),
                    preferred_element_type=jnp.float32) + b_ref[0, 0]
        m = jnp.max(a, axis=0, keepdims=True)
        e = jnp.exp(a - m)
        wts = e / jnp.sum(e, axis=0, keepdims=True)
        pooled = jnp.sum(wts * emb, axis=0, keepdims=True)
        n = jnp.sqrt(jnp.sum(pooled * pooled, axis=1, keepdims=True))
        out_ref[b:b + 1, :] = pooled / jnp.maximum(n, 1e-12)


def _compute_sset(symsT, sym_emb, attn_w, attn_b2):
    return pl.pallas_call(
        _sset_body,
        out_shape=jax.ShapeDtypeStruct((BATCH, DIM), jnp.float32),
    )(symsT, sym_emb, attn_w, attn_b2)


def _score_body(sset_ref, demb_ref, rs_ref, out_ref, acc_ref):
    i = pl.program_id(0)
    x = demb_ref[...]
    n = jnp.sqrt(jnp.sum(x * x, axis=1, keepdims=True))
    xn = x / jnp.maximum(n, 1e-12)
    dn = (((1,), (1,)), ((), ()))
    s2 = lax.dot_general(sset_ref[...].astype(jnp.bfloat16),
                         xn.astype(jnp.bfloat16), dn,
                         preferred_element_type=jnp.float32)
    sc = jnp.clip(s2, -10.0, 10.0)
    prob = jax.nn.sigmoid(sc)
    s = jnp.where(prob > 0.5, prob, 0.0)
    rid = lax.broadcasted_iota(jnp.int32, (BATCH, CP), 1) + i * CP
    s = jnp.where((rid >= 1) & (rid <= NDRUG), s, 0.0)
    out_ref[...] = s
    colsum = jnp.sum(s, axis=0, keepdims=True)
    part = jnp.sum(colsum * rs_ref[0])

    @pl.when(i == 0)
    def _():
        acc_ref[...] = jnp.zeros((1, 1), jnp.float32)

    acc_ref[...] = acc_ref[...] + part


def _score_call(s_set, drug_emb, rs_pad):
    return pl.pallas_call(
        _score_body,
        grid=(NB,),
        in_specs=[
            pl.BlockSpec((BATCH, DIM), lambda i: (0, 0)),
            pl.BlockSpec((CP, DIM), lambda i: (i, 0)),
            pl.BlockSpec((1, 1, CP), lambda i: (i, 0, 0)),
        ],
        out_specs=[
            pl.BlockSpec((BATCH, CP), lambda i: (0, i)),
            pl.BlockSpec((1, 1), lambda i: (0, 0)),
        ],
        out_shape=[
            jax.ShapeDtypeStruct((BATCH, NPAD), jnp.float32),
            jax.ShapeDtypeStruct((1, 1), jnp.float32),
        ],
    )(s_set, drug_emb, rs_pad)


NW = 32
RW = 496
GW = 16
NGW = RW // GW
TOTP = NW * RW * 128
SLAB = 62528
NBINS = 16 * SLAB


def _hist_body(idx_hbm, val_hbm, out_hbm, idx_v, val_v, zbuf, hist):
    c = lax.axis_index("c")
    s = lax.axis_index("s")
    w = s * 2 + c

    def zb(i, carry):
        zbuf[pl.ds(i * 16, 16)] = jnp.zeros((16,), jnp.float32)
        return carry

    lax.fori_loop(0, SLAB // 16, zb, 0)
    pltpu.sync_copy(zbuf, hist.at[pl.ds(s * SLAB, SLAB)])
    plsc.subcore_barrier()

    def grp(g, carry):
        pltpu.sync_copy(idx_hbm.at[w, pl.ds(g * GW, GW)], idx_v)
        pltpu.sync_copy(val_hbm.at[w, pl.ds(g * GW, GW)], val_v)
        for j in range(GW):
            pltpu.sync_copy(val_v.at[j], hist.at[idx_v.at[j]], add=True)
        return carry

    lax.fori_loop(0, NGW, grp, 0)
    plsc.subcore_barrier()
    pltpu.sync_copy(hist.at[pl.ds(s * SLAB, SLAB)], zbuf)
    pltpu.sync_copy(zbuf, out_hbm.at[c, s])


_hist_call = functools.partial(
    pl.kernel,
    out_type=jax.ShapeDtypeStruct((2, 16, SLAB), jnp.float32),
    mesh=plsc.VectorSubcoreMesh(core_axis_name="c", subcore_axis_name="s"),
    scratch_types=[
        pltpu.VMEM((GW, 128), jnp.int32),
        pltpu.VMEM((GW, 128), jnp.float32),
        pltpu.VMEM((SLAB,), jnp.float32),
        pltpu.VMEM_SHARED((NBINS,), jnp.float32),
    ],
)(_hist_body)


def _row_sums_shifted(ddi_indices, ddi_values):
    idxp = jnp.pad(ddi_indices[0] + 1, (0, TOTP - ddi_indices.shape[1]))
    valp = jnp.pad(ddi_values, (0, TOTP - ddi_values.shape[0]))
    parts = _hist_call(idxp.reshape(NW, RW, 128), valp.reshape(NW, RW, 128))
    return (parts[0] + parts[1]).reshape(NBINS)


def kernel(syms, drugs, similar_idx, sym_emb, drug_emb, attn_w, attn_b,
           ddi_indices, ddi_values):
    del drugs, similar_idx
    s_set = _compute_sset(syms.T, sym_emb, attn_w, attn_b.reshape(1, 1))
    rs = _row_sums_shifted(ddi_indices, ddi_values)
    rs_pad = jnp.pad(rs, (0, NPAD - NBINS)).reshape(NB, 1, CP)
    out_full, acc = _score_call(s_set, drug_emb, rs_pad)
    scores = lax.slice(out_full, (0, 1), (BATCH, NDRUG + 1))
    batch_neg = 1e-6 * acc[0, 0]
    scores_aug = jnp.float32(0.0)
    return (scores, scores_aug, batch_neg)


# speedup vs baseline: 7.0760x; 1.0120x over previous
import functools

import jax
import jax.numpy as jnp
from jax import lax
from jax.experimental import pallas as pl
from jax.experimental.pallas import tpu as pltpu
from jax.experimental.pallas import tpu_sc as plsc

NSYM = 1000
NDRUG = 1000000
DIM = 32
BATCH = 2
SEQ = 64
CP = 32256
NB = 32
NPAD = CP * NB


def _sset_body(symsT_ref, semb_ref, w_ref, b_ref, out_ref):
    for b in range(BATCH):
        ids = symsT_ref[:, b:b + 1]
        iota = lax.broadcasted_iota(jnp.int32, (SEQ, NSYM + 1), 1)
        oh = (iota == ids).astype(jnp.float32)
        emb = jnp.dot(oh, semb_ref[...], precision=lax.Precision.HIGHEST,
                      preferred_element_type=jnp.float32)
        a = jnp.dot(emb.astype(jnp.bfloat16), w_ref[...].astype(jnp.bfloat16),
                    preferred_element_type=jnp.float32) + b_ref[0, 0]
        m = jnp.max(a, axis=0, keepdims=True)
        e = jnp.exp(a - m)
        wts = e / jnp.sum(e, axis=0, keepdims=True)
        pooled = jnp.sum(wts * emb, axis=0, keepdims=True)
        n = jnp.sqrt(jnp.sum(pooled * pooled, axis=1, keepdims=True))
        out_ref[b:b + 1, :] = pooled / jnp.maximum(n, 1e-12)


def _compute_sset(symsT, sym_emb, attn_w, attn_b2):
    return pl.pallas_call(
        _sset_body,
        out_shape=jax.ShapeDtypeStruct((BATCH, DIM), jnp.float32),
    )(symsT, sym_emb, attn_w, attn_b2)


def _score_body(sset_ref, demb_ref, rs_ref, out_ref, acc_ref):
    i = pl.program_id(0)
    x = demb_ref[...]
    n = jnp.sqrt(jnp.sum(x * x, axis=1, keepdims=True))
    xn = x / jnp.maximum(n, 1e-12)
    dn = (((1,), (1,)), ((), ()))
    s2 = lax.dot_general(sset_ref[...].astype(jnp.bfloat16),
                         xn.astype(jnp.bfloat16), dn,
                         preferred_element_type=jnp.float32)
    sc = jnp.clip(s2, -10.0, 10.0)
    prob = jax.nn.sigmoid(sc)
    s = jnp.where(prob > 0.5, prob, 0.0)
    rid = lax.broadcasted_iota(jnp.int32, (BATCH, CP), 1) + i * CP
    s = jnp.where((rid >= 1) & (rid <= NDRUG), s, 0.0)
    out_ref[...] = s
    colsum = jnp.sum(s, axis=0, keepdims=True)
    part = jnp.sum(colsum * rs_ref[0])

    @pl.when(i == 0)
    def _():
        acc_ref[...] = jnp.zeros((1, 1), jnp.float32)

    acc_ref[...] = acc_ref[...] + part


def _score_call(s_set, drug_emb, rs_pad):
    return pl.pallas_call(
        _score_body,
        grid=(NB,),
        in_specs=[
            pl.BlockSpec((BATCH, DIM), lambda i: (0, 0)),
            pl.BlockSpec((CP, DIM), lambda i: (i, 0)),
            pl.BlockSpec((1, 1, CP), lambda i: (i, 0, 0)),
        ],
        out_specs=[
            pl.BlockSpec((BATCH, CP), lambda i: (0, i)),
            pl.BlockSpec((1, 1), lambda i: (0, 0)),
        ],
        out_shape=[
            jax.ShapeDtypeStruct((BATCH, NPAD), jnp.float32),
            jax.ShapeDtypeStruct((1, 1), jnp.float32),
        ],
    )(s_set, drug_emb, rs_pad)


NW = 32
RW = 496
GW = 16
NGW = RW // GW
TOTP = NW * RW * 128
SLAB = 62528
NBINS = 16 * SLAB


def _hist_body(idx_hbm, val_hbm, out_hbm, idx_v, val_v, zbuf, hist):
    c = lax.axis_index("c")
    s = lax.axis_index("s")
    w = s * 2 + c

    def zb(i, carry):
        zbuf[pl.ds(i * 16, 16)] = jnp.zeros((16,), jnp.float32)
        return carry

    lax.fori_loop(0, SLAB // 16, zb, 0)
    pltpu.sync_copy(zbuf, hist.at[pl.ds(s * SLAB, SLAB)])
    plsc.subcore_barrier()

    def grp(g, carry):
        pltpu.sync_copy(idx_hbm.at[w, pl.ds(g * GW, GW)], idx_v)
        pltpu.sync_copy(val_hbm.at[w, pl.ds(g * GW, GW)], val_v)
        for j in range(GW):
            pltpu.sync_copy(val_v.at[j], hist.at[idx_v.at[j]], add=True)
        return carry

    lax.fori_loop(0, NGW, grp, 0)
    plsc.subcore_barrier()
    pltpu.sync_copy(hist.at[pl.ds(s * SLAB, SLAB)], zbuf)
    pltpu.sync_copy(zbuf, out_hbm.at[c, s])


_hist_call = functools.partial(
    pl.kernel,
    out_type=jax.ShapeDtypeStruct((2, 16, SLAB), jnp.float32),
    mesh=plsc.VectorSubcoreMesh(core_axis_name="c", subcore_axis_name="s"),
    scratch_types=[
        pltpu.VMEM((GW, 128), jnp.int32),
        pltpu.VMEM((GW, 128), jnp.float32),
        pltpu.VMEM((SLAB,), jnp.float32),
        pltpu.VMEM_SHARED((NBINS,), jnp.float32),
    ],
)(_hist_body)


def _row_sums_shifted(ddi_indices, ddi_values):
    idxp = jnp.pad(ddi_indices[0] + 1, (0, TOTP - ddi_indices.shape[1]))
    valp = jnp.pad(ddi_values, (0, TOTP - ddi_values.shape[0]))
    parts = _hist_call(idxp.reshape(NW, RW, 128), valp.reshape(NW, RW, 128))
    return (parts[0] + parts[1]).reshape(NBINS)


def kernel(syms, drugs, similar_idx, sym_emb, drug_emb, attn_w, attn_b,
           ddi_indices, ddi_values):
    del drugs, similar_idx
    s_set = _compute_sset(syms.T, sym_emb, attn_w, attn_b.reshape(1, 1))
    rs = _row_sums_shifted(ddi_indices, ddi_values)
    rs_pad = jnp.pad(rs, (0, NPAD - NBINS)).reshape(NB, 1, CP)
    out_full, acc = _score_call(s_set, drug_emb, rs_pad)
    scores = lax.slice(out_full, (0, 1), (BATCH, NDRUG + 1))
    batch_neg = 1e-6 * acc[0, 0]
    scores_aug = jnp.float32(0.0)
    return (scores, scores_aug, batch_neg)
